# baseline (speedup 1.0000x reference)
import functools

import jax
import jax.numpy as jnp
from jax import lax
from jax.experimental import pallas as pl
from jax.experimental.pallas import tpu as pltpu

_RADIUS = 0.2
_K = 32
_EPS = 1e-5
_TR = 1024
_TG = _TR // _K


def _fps_jax(coords, n_samples):
    B, N, _ = coords.shape
    batch_idx = jnp.arange(B)
    guess = jnp.mean(coords, axis=1, keepdims=True)
    far = jnp.argmax(jnp.sum((coords - guess) ** 2, axis=-1), axis=1).astype(jnp.int32)
    md = jnp.full((B, N), jnp.inf, dtype=coords.dtype)
    cents = jnp.zeros((B, n_samples), dtype=jnp.int32)

    def body(i, state):
        cents, md, far = state
        cents = cents.at[:, i].set(far)
        centroid = coords[batch_idx, far][:, None, :]
        d = jnp.sum((coords - centroid) ** 2, axis=-1)
        md = jnp.minimum(md, d)
        far = jnp.argmax(md, axis=1).astype(jnp.int32)
        return (cents, md, far)

    cents, _, _ = lax.fori_loop(0, n_samples, body, (cents, md, far))
    return cents


def _fps_body(p_ref, fps_ref, md_ref, *, n_samples):
    P = p_ref[...]
    nb3, N = P.shape
    B = nb3 // 3
    iotaL = lax.broadcasted_iota(jnp.int32, (B, N), 1)
    iotaM = lax.broadcasted_iota(jnp.int32, fps_ref.shape, 1)

    def dist_to(c):
        t = P - c
        t = t * t
        return t[0:B] + t[B:2 * B] + t[2 * B:3 * B]

    def argmax_rows(d):
        m = jnp.max(d, axis=1, keepdims=True)
        return jnp.min(jnp.where(d == m, iotaL, jnp.int32(N)), axis=1,
                       keepdims=True)

    guess = jnp.mean(P, axis=1, keepdims=True)
    far0 = argmax_rows(dist_to(guess))
    md_ref[...] = jnp.full((B, N), jnp.inf, jnp.float32)

    def body(i, far):
        fps_ref[...] = jnp.where(iotaM == i, jnp.broadcast_to(far, fps_ref.shape),
                                 fps_ref[...])
        far3 = jnp.concatenate([far, far, far], axis=0)
        c = jnp.sum(jnp.where(lax.broadcasted_iota(jnp.int32, (3 * B, N), 1)
                              == far3, P, 0.0), axis=1, keepdims=True)
        md = jnp.minimum(md_ref[...], dist_to(c))
        md_ref[...] = md
        return argmax_rows(md)

    lax.fori_loop(0, n_samples, body, far0)


def _mlp1_body(x_ref, nc_ref, cf_ref, wc_ref, wa_ref, ws_ref,
               u1_ref, i_ref, s1_ref, si_ref):
    step = pl.program_id(0)

    @pl.when(step == 0)
    def _init():
        s1_ref[...] = jnp.zeros_like(s1_ref)
        si_ref[...] = jnp.zeros_like(si_ref)

    u = jnp.dot(x_ref[...], wc_ref[...], preferred_element_type=jnp.float32)
    c = jnp.dot(nc_ref[...], wa_ref[...], preferred_element_type=jnp.float32)
    u = (u.reshape(_TG, _K, u.shape[-1]) - c[:, None, :]).reshape(u.shape)
    u1_ref[...] = u
    it = jnp.dot(cf_ref[...], ws_ref[...], preferred_element_type=jnp.float32)
    i_ref[...] = it
    s1_ref[0:1, :] += jnp.sum(u, axis=0, keepdims=True)
    s1_ref[1:2, :] += jnp.sum(u * u, axis=0, keepdims=True)
    si_ref[0:1, :] += jnp.sum(it, axis=0, keepdims=True)
    si_ref[1:2, :] += jnp.sum(it * it, axis=0, keepdims=True)


def _mlp2_body(u1_ref, s1_ref, w2_ref, gb1_ref, n_ref, r_ref, s2_ref):
    step = pl.program_id(0)

    @pl.when(step == 0)
    def _init():
        s2_ref[...] = jnp.zeros_like(s2_ref)

    n = n_ref[0]
    mean = s1_ref[0:1, :] / n
    var = s1_ref[1:2, :] / n - mean * mean
    inv = lax.rsqrt(var + _EPS)
    scale = gb1_ref[0:1, :] * inv
    shift = gb1_ref[1:2, :] - mean * scale
    h = jnp.maximum(u1_ref[...] * scale + shift, 0.0)
    u2 = jnp.dot(h, w2_ref[...], preferred_element_type=jnp.float32)
    s2_ref[0:1, :] += jnp.sum(u2, axis=0, keepdims=True)
    s2_ref[1:2, :] += jnp.sum(u2 * u2, axis=0, keepdims=True)
    r_ref[...] = jnp.max(u2.reshape(_TG, _K, u2.shape[-1]), axis=1)


def _final_body(r_ref, i_ref, s2_ref, si_ref, p_ref, n_ref, o_ref):
    n2 = n_ref[0]
    ni = n_ref[1]
    m2 = s2_ref[0:1, :] / n2
    v2 = s2_ref[1:2, :] / n2 - m2 * m2
    inv2 = lax.rsqrt(v2 + _EPS)
    sc2 = p_ref[0:1, :] * inv2
    sh2 = p_ref[1:2, :] - m2 * sc2
    mi = si_ref[0:1, :] / ni
    vi = si_ref[1:2, :] / ni - mi * mi
    invi = lax.rsqrt(vi + _EPS)
    sci = p_ref[2:3, :] * invi
    shi = p_ref[3:4, :] - mi * sci
    red = jnp.maximum(r_ref[...] * sc2 + sh2, 0.0)
    idn = i_ref[...] * sci + shi
    o_ref[...] = jnp.maximum(red + idn, 0.0)


def kernel(coords, feats, W1, g1, b1, W2, g2, b2, Ws, gs, bs):
    B, N, _ = coords.shape
    M = N // 4
    C = feats.shape[1]
    MID = W1.shape[0]
    OUT = W2.shape[0]
    R = B * M * _K
    NG = B * M

    P = jnp.transpose(coords, (2, 0, 1)).reshape(3 * B, N)
    fps_idx = pl.pallas_call(
        functools.partial(_fps_body, n_samples=M),
        grid=(1,),
        in_specs=[pl.BlockSpec((3 * B, N), lambda i: (0, 0))],
        out_specs=pl.BlockSpec((B, M), lambda i: (0, 0)),
        out_shape=jax.ShapeDtypeStruct((B, M), jnp.int32),
        scratch_shapes=[pltpu.VMEM((B, N), jnp.float32)],
    )(P)

    featsT = jnp.transpose(feats, (0, 2, 1))
    pad = 80 - (3 + C)
    table = jnp.concatenate(
        [coords, featsT, jnp.zeros((B, N, pad), jnp.float32)], axis=-1
    ).reshape(B * N, 3 + C + pad)

    base = (jnp.arange(B, dtype=jnp.int32) * N)[:, None]
    cf_idx = (fps_idx + base).reshape(-1)
    new_coords = jnp.take(coords.reshape(B * N, 3), cf_idx, axis=0).reshape(B, M, 3)

    q2 = jnp.sum(new_coords * new_coords, axis=-1)[..., :, None]
    s2 = jnp.sum(coords * coords, axis=-1)[..., None, :]
    ab = jnp.matmul(new_coords, jnp.swapaxes(coords, -1, -2))
    dist = jnp.sqrt(jnp.maximum(q2 + s2 - 2.0 * ab, 0.0))
    idx = lax.top_k(-dist, _K)[1].astype(jnp.int32)
    flat_idx = (idx + base[:, :, None]).reshape(-1)

    x = jnp.take(table, flat_idx, axis=0)
    cfeats = jnp.take(table, cf_idx, axis=0)[:, 3:3 + C]
    nc_pad = jnp.zeros((NG, 8), jnp.float32).at[:, :3].set(
        new_coords.reshape(NG, 3))

    w1a = jnp.transpose(W1[:, :3]) / _RADIUS
    Wc = jnp.zeros((3 + C + pad, MID), jnp.float32)
    Wc = Wc.at[:3].set(w1a).at[3:3 + C].set(jnp.transpose(W1[:, 3:]))
    Wa = jnp.zeros((8, MID), jnp.float32).at[:3].set(w1a)
    W2T = jnp.transpose(W2)
    WsT = jnp.transpose(Ws)

    gb1 = jnp.zeros((8, MID), jnp.float32).at[0].set(g1).at[1].set(b1)
    P = jnp.zeros((8, OUT), jnp.float32).at[0].set(g2).at[1].set(b2)\
        .at[2].set(gs).at[3].set(bs)

    grid_a = R // _TR
    u1, ident, s1, si = pl.pallas_call(
        _mlp1_body,
        grid=(grid_a,),
        in_specs=[
            pl.BlockSpec((_TR, 3 + C + pad), lambda i: (i, 0)),
            pl.BlockSpec((_TG, 8), lambda i: (i, 0)),
            pl.BlockSpec((_TG, C), lambda i: (i, 0)),
            pl.BlockSpec((3 + C + pad, MID), lambda i: (0, 0)),
            pl.BlockSpec((8, MID), lambda i: (0, 0)),
            pl.BlockSpec((C, OUT), lambda i: (0, 0)),
        ],
        out_specs=[
            pl.BlockSpec((_TR, MID), lambda i: (i, 0)),
            pl.BlockSpec((_TG, OUT), lambda i: (i, 0)),
            pl.BlockSpec((8, MID), lambda i: (0, 0)),
            pl.BlockSpec((8, OUT), lambda i: (0, 0)),
        ],
        out_shape=[
            jax.ShapeDtypeStruct((R, MID), jnp.float32),
            jax.ShapeDtypeStruct((NG, OUT), jnp.float32),
            jax.ShapeDtypeStruct((8, MID), jnp.float32),
            jax.ShapeDtypeStruct((8, OUT), jnp.float32),
        ],
    )(x, nc_pad, cfeats, Wc, Wa, WsT)

    counts = jnp.array([float(R), float(NG)], jnp.float32)

    rmax, s2s = pl.pallas_call(
        _mlp2_body,
        grid=(grid_a,),
        in_specs=[
            pl.BlockSpec((_TR, MID), lambda i: (i, 0)),
            pl.BlockSpec((8, MID), lambda i: (0, 0)),
            pl.BlockSpec((MID, OUT), lambda i: (0, 0)),
            pl.BlockSpec((8, MID), lambda i: (0, 0)),
            pl.BlockSpec(memory_space=pltpu.SMEM),
        ],
        out_specs=[
            pl.BlockSpec((_TG, OUT), lambda i: (i, 0)),
            pl.BlockSpec((8, OUT), lambda i: (0, 0)),
        ],
        out_shape=[
            jax.ShapeDtypeStruct((NG, OUT), jnp.float32),
            jax.ShapeDtypeStruct((8, OUT), jnp.float32),
        ],
    )(u1, s1, W2T, gb1, counts)

    out2d = pl.pallas_call(
        _final_body,
        grid=(B,),
        in_specs=[
            pl.BlockSpec((M, OUT), lambda i: (i, 0)),
            pl.BlockSpec((M, OUT), lambda i: (i, 0)),
            pl.BlockSpec((8, OUT), lambda i: (0, 0)),
            pl.BlockSpec((8, OUT), lambda i: (0, 0)),
            pl.BlockSpec((8, OUT), lambda i: (0, 0)),
            pl.BlockSpec(memory_space=pltpu.SMEM),
        ],
        out_specs=pl.BlockSpec((M, OUT), lambda i: (i, 0)),
        out_shape=jax.ShapeDtypeStruct((NG, OUT), jnp.float32),
    )(rmax, ident, s2s, si, P, counts)

    out = jnp.transpose(out2d.reshape(B, M, OUT), (0, 2, 1))
    return new_coords, out


# speedup vs baseline: 3.1198x; 3.1198x over previous
import functools

import jax
import jax.numpy as jnp
from jax import lax
from jax.experimental import pallas as pl
from jax.experimental.pallas import tpu as pltpu
from jax.experimental.pallas import tpu_sc as plsc

_RADIUS = 0.2
_K = 32
_EPS = 1e-5
_TR = 1024
_TG = _TR // _K
_TAUS = [0.002 * (2.0 ** j) for j in range(12)]
_NW = 32
_QPW = 256


def _fps_jax(coords, n_samples):
    B, N, _ = coords.shape
    batch_idx = jnp.arange(B)
    guess = jnp.mean(coords, axis=1, keepdims=True)
    far = jnp.argmax(jnp.sum((coords - guess) ** 2, axis=-1), axis=1).astype(jnp.int32)
    md = jnp.full((B, N), jnp.inf, dtype=coords.dtype)
    cents = jnp.zeros((B, n_samples), dtype=jnp.int32)

    def body(i, state):
        cents, md, far = state
        cents = cents.at[:, i].set(far)
        centroid = coords[batch_idx, far][:, None, :]
        d = jnp.sum((coords - centroid) ** 2, axis=-1)
        md = jnp.minimum(md, d)
        far = jnp.argmax(md, axis=1).astype(jnp.int32)
        return (cents, md, far)

    cents, _, _ = lax.fori_loop(0, n_samples, body, (cents, md, far))
    return cents


def _fps_body(p_ref, fps_ref, md_ref, *, n_samples):
    P = p_ref[...]
    nb3, N = P.shape
    B = nb3 // 3
    iotaL = lax.broadcasted_iota(jnp.int32, (B, N), 1)
    iotaM = lax.broadcasted_iota(jnp.int32, fps_ref.shape, 1)

    def dist_to(c):
        t = P - c
        t = t * t
        return t[0:B] + t[B:2 * B] + t[2 * B:3 * B]

    def argmax_rows(d):
        m = jnp.max(d, axis=1, keepdims=True)
        return jnp.min(jnp.where(d == m, iotaL, jnp.int32(N)), axis=1,
                       keepdims=True)

    guess = jnp.mean(P, axis=1, keepdims=True)
    far0 = argmax_rows(dist_to(guess))
    md_ref[...] = jnp.full((B, N), jnp.inf, jnp.float32)

    def body(i, far):
        fps_ref[...] = jnp.where(iotaM == i, jnp.broadcast_to(far, fps_ref.shape),
                                 fps_ref[...])
        far3 = jnp.concatenate([far, far, far], axis=0)
        c = jnp.sum(jnp.where(lax.broadcasted_iota(jnp.int32, (3 * B, N), 1)
                              == far3, P, 0.0), axis=1, keepdims=True)
        md = jnp.minimum(md_ref[...], dist_to(c))
        md_ref[...] = md
        return argmax_rows(md)

    lax.fori_loop(0, n_samples, body, far0)


def _score_body(q_ref, s_ref, s2_ref, t_ref, d2_ref, cnt_ref):
    ab = jnp.dot(q_ref[...], s_ref[0], preferred_element_type=jnp.float32)
    d2 = s2_ref[0] - 2.0 * ab
    d2_ref[...] = d2
    Mq = d2.shape[0]
    lane = lax.broadcasted_iota(jnp.int32, (Mq, 128), 1)
    acc = jnp.zeros((Mq, 128), jnp.float32)
    for j in range(len(_TAUS)):
        cj = jnp.sum((d2 < t_ref[:, j:j + 1]).astype(jnp.float32), axis=1,
                     keepdims=True)
        acc = acc + jnp.where(lane == j, cj, 0.0)
    cnt_ref[...] = acc


def _make_knn_sc(B, N, M, NG):
    mesh = plsc.VectorSubcoreMesh(core_axis_name="c", subcore_axis_name="s")

    @functools.partial(
        pl.kernel, mesh=mesh,
        compiler_params=pltpu.CompilerParams(needs_layout_passes=False),
        out_type=jax.ShapeDtypeStruct((NG * _K,), jnp.int32),
        scratch_types=[
            pltpu.VMEM((N,), jnp.float32),
            pltpu.VMEM((N,), jnp.float32),
            pltpu.VMEM((_QPW + 16,), jnp.float32),
            pltpu.VMEM((N + 32,), jnp.float32),
            pltpu.VMEM((N + 32,), jnp.int32),
            pltpu.VMEM((_QPW * _K,), jnp.int32),
            pltpu.SemaphoreType.DMA,
            pltpu.SemaphoreType.DMA,
        ],
    )
    def knn(d2_hbm, tau_hbm, out_hbm, bufA, bufB, tau_v, candD, candI, obuf,
            semA, semB):
        wid = lax.axis_index("s") * 2 + lax.axis_index("c")
        b = wid // (M // _QPW)
        base = wid * _QPW
        pltpu.sync_copy(tau_hbm.at[wid], tau_v.at[pl.ds(0, _QPW)])

        iota16 = lax.iota(jnp.int32, 16)
        inf16 = jnp.full((16,), jnp.inf, jnp.float32)
        zero16i = jnp.zeros((16,), jnp.int32)
        zf = jnp.zeros((16,), jnp.float32)
        gbase = b * N

        def cpA(q):
            return pltpu.make_async_copy(d2_hbm.at[base + q], bufA, semA)

        def cpB(q):
            return pltpu.make_async_copy(d2_hbm.at[base + q], bufB, semB)

        cpA(0).start()
        cpB(1).start()

        def per_pair(g, _):
            q0 = 2 * g
            cpA(q0).wait()
            _one_query(q0, bufA)

            @pl.when(q0 + 2 < _QPW)
            def _():
                cpA(q0 + 2).start()
            cpB(q0 + 1).wait()
            _one_query(q0 + 1, bufB)

            @pl.when(q0 + 3 < _QPW)
            def _():
                cpB(q0 + 3).start()
            return 0

        def _one_query(q, dbuf):
            tauv = plsc.load_gather(tau_v, [jnp.zeros((16,), jnp.int32) + q])

            candD[pl.ds(0, 16)] = inf16
            candD[pl.ds(16, 16)] = inf16
            candD[pl.ds(32, 16)] = inf16
            candI[pl.ds(0, 16)] = zero16i
            candI[pl.ds(16, 16)] = zero16i
            candI[pl.ds(32, 16)] = zero16i

            def scan_body(v, off):
                for u in range(4):
                    s = (v * 4 + u) * 16
                    d = dbuf[pl.ds(s, 16)]
                    m = d < tauv
                    pos = (off - 1) + plsc.cumsum(m.astype(jnp.int32))
                    plsc.store_scatter(candD, [pos], d, mask=m)
                    plsc.store_scatter(candI, [pos], iota16 + s, mask=m)
                    off = off + plsc.all_reduce_population_count(m)[0]
                return off

            off = lax.fori_loop(0, N // 64, scan_body, jnp.int32(0),
                                unroll=False)
            candD[pl.ds(off, 16)] = inf16
            candI[pl.ds(off, 16)] = zero16i
            candD[pl.ds(off + 16, 16)] = inf16
            candI[pl.ds(off + 16, 16)] = zero16i

            def merge_body(t, carry):
                r0k, r0v, r1k, r1v = carry
                ck = candD[pl.ds(t * 16, 16)]
                cv = candI[pl.ds(t * 16, 16)]
                ck, cv = plsc.sort_key_val(ck, cv)
                rck = lax.rev(ck, (0,))
                rcv = lax.rev(cv, (0,))
                m = rck < r1k
                sk = jnp.where(m, rck, r1k)
                sv = jnp.where(m, rcv, r1v)
                sk, sv = plsc.sort_key_val(sk, sv)
                rsk = lax.rev(sk, (0,))
                rsv = lax.rev(sv, (0,))
                m2 = r0k < rsk
                lk = jnp.where(m2, r0k, rsk)
                lv = jnp.where(m2, r0v, rsv)
                hk = jnp.where(m2, rsk, r0k)
                hv = jnp.where(m2, rsv, r0v)
                r0k, r0v = plsc.sort_key_val(lk, lv)
                r1k, r1v = plsc.sort_key_val(hk, hv)
                return r0k, r0v, r1k, r1v

            nv = (off + 15) // 16
            _, r0v, _, r1v = lax.fori_loop(
                0, nv, merge_body, (inf16, zero16i, inf16, zero16i))
            obuf[pl.ds(q * _K, 16)] = r0v + gbase
            obuf[pl.ds(q * _K + 16, 16)] = r1v + gbase

        lax.fori_loop(0, _QPW // 2, per_pair, 0, unroll=False)
        pltpu.sync_copy(obuf, out_hbm.at[pl.ds(wid * (_QPW * _K), _QPW * _K)])

    return knn


def _mlp1_body(x_ref, nc_ref, cf_ref, wc_ref, wa_ref, ws_ref,
               u1_ref, i_ref, s1_ref, si_ref):
    step = pl.program_id(0)

    @pl.when(step == 0)
    def _init():
        s1_ref[...] = jnp.zeros_like(s1_ref)
        si_ref[...] = jnp.zeros_like(si_ref)

    u = jnp.dot(x_ref[...], wc_ref[...], preferred_element_type=jnp.float32)
    c = jnp.dot(nc_ref[...], wa_ref[...], preferred_element_type=jnp.float32)
    u = (u.reshape(_TG, _K, u.shape[-1]) - c[:, None, :]).reshape(u.shape)
    u1_ref[...] = u
    it = jnp.dot(cf_ref[...], ws_ref[...], preferred_element_type=jnp.float32)
    i_ref[...] = it
    s1_ref[0:1, :] += jnp.sum(u, axis=0, keepdims=True)
    s1_ref[1:2, :] += jnp.sum(u * u, axis=0, keepdims=True)
    si_ref[0:1, :] += jnp.sum(it, axis=0, keepdims=True)
    si_ref[1:2, :] += jnp.sum(it * it, axis=0, keepdims=True)


def _mlp2_body(u1_ref, s1_ref, w2_ref, gb1_ref, n_ref, r_ref, s2_ref):
    step = pl.program_id(0)

    @pl.when(step == 0)
    def _init():
        s2_ref[...] = jnp.zeros_like(s2_ref)

    n = n_ref[0]
    mean = s1_ref[0:1, :] / n
    var = s1_ref[1:2, :] / n - mean * mean
    inv = lax.rsqrt(var + _EPS)
    scale = gb1_ref[0:1, :] * inv
    shift = gb1_ref[1:2, :] - mean * scale
    h = jnp.maximum(u1_ref[...] * scale + shift, 0.0)
    u2 = jnp.dot(h, w2_ref[...], preferred_element_type=jnp.float32)
    s2_ref[0:1, :] += jnp.sum(u2, axis=0, keepdims=True)
    s2_ref[1:2, :] += jnp.sum(u2 * u2, axis=0, keepdims=True)
    r_ref[...] = jnp.max(u2.reshape(_TG, _K, u2.shape[-1]), axis=1)


def _final_body(r_ref, i_ref, s2_ref, si_ref, p_ref, n_ref, o_ref):
    n2 = n_ref[0]
    ni = n_ref[1]
    m2 = s2_ref[0:1, :] / n2
    v2 = s2_ref[1:2, :] / n2 - m2 * m2
    inv2 = lax.rsqrt(v2 + _EPS)
    sc2 = p_ref[0:1, :] * inv2
    sh2 = p_ref[1:2, :] - m2 * sc2
    mi = si_ref[0:1, :] / ni
    vi = si_ref[1:2, :] / ni - mi * mi
    invi = lax.rsqrt(vi + _EPS)
    sci = p_ref[2:3, :] * invi
    shi = p_ref[3:4, :] - mi * sci
    red = jnp.maximum(r_ref[...] * sc2 + sh2, 0.0)
    idn = i_ref[...] * sci + shi
    o_ref[...] = jnp.maximum(red + idn, 0.0)


def kernel(coords, feats, W1, g1, b1, W2, g2, b2, Ws, gs, bs):
    B, N, _ = coords.shape
    M = N // 4
    C = feats.shape[1]
    MID = W1.shape[0]
    OUT = W2.shape[0]
    R = B * M * _K
    NG = B * M

    P = jnp.transpose(coords, (2, 0, 1)).reshape(3 * B, N)
    fps_idx = pl.pallas_call(
        functools.partial(_fps_body, n_samples=M),
        grid=(1,),
        in_specs=[pl.BlockSpec((3 * B, N), lambda i: (0, 0))],
        out_specs=pl.BlockSpec((B, M), lambda i: (0, 0)),
        out_shape=jax.ShapeDtypeStruct((B, M), jnp.int32),
        scratch_shapes=[pltpu.VMEM((B, N), jnp.float32)],
    )(P)

    featsT = jnp.transpose(feats, (0, 2, 1))
    pad = 80 - (3 + C)
    table = jnp.concatenate(
        [coords, featsT, jnp.zeros((B, N, pad), jnp.float32)], axis=-1
    ).reshape(B * N, 3 + C + pad)

    base = (jnp.arange(B, dtype=jnp.int32) * N)[:, None]
    cf_idx = (fps_idx + base).reshape(-1)
    new_coords = jnp.take(coords.reshape(B * N, 3), cf_idx, axis=0).reshape(B, M, 3)

    q2f = jnp.sum(new_coords * new_coords, axis=-1).reshape(NG)
    s2 = jnp.sum(coords * coords, axis=-1)
    coordsB = jnp.transpose(coords, (0, 2, 1))
    Qmat = jnp.concatenate([
        new_coords.reshape(NG, 3),
        jnp.zeros((NG, 5), jnp.float32)], axis=1)
    Smat = jnp.concatenate([
        coordsB,
        jnp.zeros((B, 5, N), jnp.float32)], axis=1)
    taus = jnp.array(_TAUS, jnp.float32)
    t12 = taus[None, :] - q2f[:, None]
    t_in = jnp.concatenate([t12, jnp.zeros((NG, 116), jnp.float32)], axis=1)
    d2mat, cnt = pl.pallas_call(
        _score_body,
        grid=(B,),
        in_specs=[
            pl.BlockSpec((M, 8), lambda i: (i, 0)),
            pl.BlockSpec((1, 8, N), lambda i: (i, 0, 0)),
            pl.BlockSpec((1, 1, N), lambda i: (i, 0, 0)),
            pl.BlockSpec((M, 128), lambda i: (i, 0)),
        ],
        out_specs=[
            pl.BlockSpec((M, N), lambda i: (i, 0)),
            pl.BlockSpec((M, 128), lambda i: (i, 0)),
        ],
        out_shape=[
            jax.ShapeDtypeStruct((NG, N), jnp.float32),
            jax.ShapeDtypeStruct((NG, 128), jnp.float32),
        ],
    )(Qmat, Smat, s2[:, None, :], t_in)
    tau_sel = jnp.min(jnp.where(cnt[:, :len(_TAUS)] >= 33.0, t12, jnp.inf),
                      axis=1)
    flat_idx = _make_knn_sc(B, N, M, NG)(
        d2mat, tau_sel.reshape(_NW, _QPW))

    x = jnp.take(table, flat_idx, axis=0)
    cfeats = jnp.take(table, cf_idx, axis=0)[:, 3:3 + C]
    nc_pad = jnp.zeros((NG, 8), jnp.float32).at[:, :3].set(
        new_coords.reshape(NG, 3))

    w1a = jnp.transpose(W1[:, :3]) / _RADIUS
    Wc = jnp.zeros((3 + C + pad, MID), jnp.float32)
    Wc = Wc.at[:3].set(w1a).at[3:3 + C].set(jnp.transpose(W1[:, 3:]))
    Wa = jnp.zeros((8, MID), jnp.float32).at[:3].set(w1a)
    W2T = jnp.transpose(W2)
    WsT = jnp.transpose(Ws)

    gb1 = jnp.zeros((8, MID), jnp.float32).at[0].set(g1).at[1].set(b1)
    P = jnp.zeros((8, OUT), jnp.float32).at[0].set(g2).at[1].set(b2)\
        .at[2].set(gs).at[3].set(bs)

    grid_a = R // _TR
    u1, ident, s1, si = pl.pallas_call(
        _mlp1_body,
        grid=(grid_a,),
        in_specs=[
            pl.BlockSpec((_TR, 3 + C + pad), lambda i: (i, 0)),
            pl.BlockSpec((_TG, 8), lambda i: (i, 0)),
            pl.BlockSpec((_TG, C), lambda i: (i, 0)),
            pl.BlockSpec((3 + C + pad, MID), lambda i: (0, 0)),
            pl.BlockSpec((8, MID), lambda i: (0, 0)),
            pl.BlockSpec((C, OUT), lambda i: (0, 0)),
        ],
        out_specs=[
            pl.BlockSpec((_TR, MID), lambda i: (i, 0)),
            pl.BlockSpec((_TG, OUT), lambda i: (i, 0)),
            pl.BlockSpec((8, MID), lambda i: (0, 0)),
            pl.BlockSpec((8, OUT), lambda i: (0, 0)),
        ],
        out_shape=[
            jax.ShapeDtypeStruct((R, MID), jnp.float32),
            jax.ShapeDtypeStruct((NG, OUT), jnp.float32),
            jax.ShapeDtypeStruct((8, MID), jnp.float32),
            jax.ShapeDtypeStruct((8, OUT), jnp.float32),
        ],
    )(x, nc_pad, cfeats, Wc, Wa, WsT)

    counts = jnp.array([float(R), float(NG)], jnp.float32)

    rmax, s2s = pl.pallas_call(
        _mlp2_body,
        grid=(grid_a,),
        in_specs=[
            pl.BlockSpec((_TR, MID), lambda i: (i, 0)),
            pl.BlockSpec((8, MID), lambda i: (0, 0)),
            pl.BlockSpec((MID, OUT), lambda i: (0, 0)),
            pl.BlockSpec((8, MID), lambda i: (0, 0)),
            pl.BlockSpec(memory_space=pltpu.SMEM),
        ],
        out_specs=[
            pl.BlockSpec((_TG, OUT), lambda i: (i, 0)),
            pl.BlockSpec((8, OUT), lambda i: (0, 0)),
        ],
        out_shape=[
            jax.ShapeDtypeStruct((NG, OUT), jnp.float32),
            jax.ShapeDtypeStruct((8, OUT), jnp.float32),
        ],
    )(u1, s1, W2T, gb1, counts)

    out2d = pl.pallas_call(
        _final_body,
        grid=(B,),
        in_specs=[
            pl.BlockSpec((M, OUT), lambda i: (i, 0)),
            pl.BlockSpec((M, OUT), lambda i: (i, 0)),
            pl.BlockSpec((8, OUT), lambda i: (0, 0)),
            pl.BlockSpec((8, OUT), lambda i: (0, 0)),
            pl.BlockSpec((8, OUT), lambda i: (0, 0)),
            pl.BlockSpec(memory_space=pltpu.SMEM),
        ],
        out_specs=pl.BlockSpec((M, OUT), lambda i: (i, 0)),
        out_shape=jax.ShapeDtypeStruct((NG, OUT), jnp.float32),
    )(rmax, ident, s2s, si, P, counts)

    out = jnp.transpose(out2d.reshape(B, M, OUT), (0, 2, 1))
    return new_coords, out


# speedup vs baseline: 3.3898x; 1.0866x over previous
import functools

import jax
import jax.numpy as jnp
from jax import lax
from jax.experimental import pallas as pl
from jax.experimental.pallas import tpu as pltpu
from jax.experimental.pallas import tpu_sc as plsc

_RADIUS = 0.2
_K = 32
_EPS = 1e-5
_TR = 1024
_TG = _TR // _K
_TAUS = [0.002 * (2.0 ** j) for j in range(12)]
_NW = 32
_QPW = 256


def _fps_jax(coords, n_samples):
    B, N, _ = coords.shape
    batch_idx = jnp.arange(B)
    guess = jnp.mean(coords, axis=1, keepdims=True)
    far = jnp.argmax(jnp.sum((coords - guess) ** 2, axis=-1), axis=1).astype(jnp.int32)
    md = jnp.full((B, N), jnp.inf, dtype=coords.dtype)
    cents = jnp.zeros((B, n_samples), dtype=jnp.int32)

    def body(i, state):
        cents, md, far = state
        cents = cents.at[:, i].set(far)
        centroid = coords[batch_idx, far][:, None, :]
        d = jnp.sum((coords - centroid) ** 2, axis=-1)
        md = jnp.minimum(md, d)
        far = jnp.argmax(md, axis=1).astype(jnp.int32)
        return (cents, md, far)

    cents, _, _ = lax.fori_loop(0, n_samples, body, (cents, md, far))
    return cents


def _fps_body(p_ref, fps_ref, md_ref, *, n_samples):
    P = p_ref[...]
    nb3, N = P.shape
    B = nb3 // 3
    iotaL = lax.broadcasted_iota(jnp.int32, (B, N), 1)
    iotaM = lax.broadcasted_iota(jnp.int32, fps_ref.shape, 1)

    def dist_to(c):
        t = P - c
        t = t * t
        return t[0:B] + t[B:2 * B] + t[2 * B:3 * B]

    def argmax_rows(d):
        m = jnp.max(d, axis=1, keepdims=True)
        return jnp.min(jnp.where(d == m, iotaL, jnp.int32(N)), axis=1,
                       keepdims=True)

    guess = jnp.mean(P, axis=1, keepdims=True)
    far0 = argmax_rows(dist_to(guess))
    md_ref[...] = jnp.full((B, N), jnp.inf, jnp.float32)

    def body(i, far):
        fps_ref[...] = jnp.where(iotaM == i, jnp.broadcast_to(far, fps_ref.shape),
                                 fps_ref[...])
        far3 = jnp.concatenate([far, far, far], axis=0)
        c = jnp.sum(jnp.where(lax.broadcasted_iota(jnp.int32, (3 * B, N), 1)
                              == far3, P, 0.0), axis=1, keepdims=True)
        md = jnp.minimum(md_ref[...], dist_to(c))
        md_ref[...] = md
        return argmax_rows(md)

    lax.fori_loop(0, n_samples, body, far0)


def _score_body(q_ref, s_ref, s2_ref, t_ref, d2_ref, cnt_ref):
    ab = jnp.dot(q_ref[...], s_ref[0], preferred_element_type=jnp.float32)
    d2 = s2_ref[0] - 2.0 * ab
    d2_ref[...] = d2
    Mq = d2.shape[0]
    lane = lax.broadcasted_iota(jnp.int32, (Mq, 128), 1)
    acc = jnp.zeros((Mq, 128), jnp.float32)
    for j in range(len(_TAUS)):
        cj = jnp.sum((d2 < t_ref[:, j:j + 1]).astype(jnp.float32), axis=1,
                     keepdims=True)
        acc = acc + jnp.where(lane == j, cj, 0.0)
    cnt_ref[...] = acc


def _make_knn_sc(B, N, M, NG):
    mesh = plsc.VectorSubcoreMesh(core_axis_name="c", subcore_axis_name="s")

    @functools.partial(
        pl.kernel, mesh=mesh,
        compiler_params=pltpu.CompilerParams(needs_layout_passes=False),
        out_type=jax.ShapeDtypeStruct((NG * _K,), jnp.int32),
        scratch_types=[
            pltpu.VMEM((N,), jnp.float32),
            pltpu.VMEM((N,), jnp.float32),
            pltpu.VMEM((_QPW + 16,), jnp.float32),
            pltpu.VMEM((N + 32,), jnp.float32),
            pltpu.VMEM((N + 32,), jnp.int32),
            pltpu.VMEM((_QPW * _K,), jnp.int32),
            pltpu.SemaphoreType.DMA,
            pltpu.SemaphoreType.DMA,
        ],
    )
    def knn(d2_hbm, tau_hbm, out_hbm, bufA, bufB, tau_v, candD, candI, obuf,
            semA, semB):
        wid = lax.axis_index("s") * 2 + lax.axis_index("c")
        b = wid // (M // _QPW)
        base = wid * _QPW
        pltpu.sync_copy(tau_hbm.at[wid], tau_v.at[pl.ds(0, _QPW)])

        iota16 = lax.iota(jnp.int32, 16)
        inf16 = jnp.full((16,), jnp.inf, jnp.float32)
        zero16i = jnp.zeros((16,), jnp.int32)
        zf = jnp.zeros((16,), jnp.float32)
        gbase = b * N

        def cpA(q):
            return pltpu.make_async_copy(d2_hbm.at[base + q], bufA, semA)

        def cpB(q):
            return pltpu.make_async_copy(d2_hbm.at[base + q], bufB, semB)

        cpA(0).start()
        cpB(1).start()

        def per_pair(g, _):
            q0 = 2 * g
            cpA(q0).wait()
            _one_query(q0, bufA)

            @pl.when(q0 + 2 < _QPW)
            def _():
                cpA(q0 + 2).start()
            cpB(q0 + 1).wait()
            _one_query(q0 + 1, bufB)

            @pl.when(q0 + 3 < _QPW)
            def _():
                cpB(q0 + 3).start()
            return 0

        def _one_query(q, dbuf):
            tauv = plsc.load_gather(tau_v, [jnp.zeros((16,), jnp.int32) + q])

            candD[pl.ds(0, 16)] = inf16
            candD[pl.ds(16, 16)] = inf16
            candD[pl.ds(32, 16)] = inf16
            candI[pl.ds(0, 16)] = zero16i
            candI[pl.ds(16, 16)] = zero16i
            candI[pl.ds(32, 16)] = zero16i

            def scan_body(v, off):
                ds, ms = [], []
                for u in range(4):
                    d = dbuf[pl.ds((v * 4 + u) * 16, 16)]
                    ds.append(d)
                    ms.append(d < tauv)
                anym = (ms[0] | ms[1]) | (ms[2] | ms[3])

                def compact(off):
                    for u in range(4):
                        m = ms[u]
                        pos = (off - 1) + plsc.cumsum(m.astype(jnp.int32))
                        plsc.store_scatter(candD, [pos], ds[u], mask=m)
                        plsc.store_scatter(candI, [pos],
                                           iota16 + (v * 4 + u) * 16, mask=m)
                        off = off + plsc.all_reduce_population_count(m)[0]
                    return off

                return lax.cond(plsc.all_reduce_population_count(anym)[0] > 0,
                                compact, lambda o: o, off)

            off = lax.fori_loop(0, N // 64, scan_body, jnp.int32(0),
                                unroll=False)
            candD[pl.ds(off, 16)] = inf16
            candI[pl.ds(off, 16)] = zero16i
            candD[pl.ds(off + 16, 16)] = inf16
            candI[pl.ds(off + 16, 16)] = zero16i

            def merge_body(t, carry):
                r0k, r0v, r1k, r1v = carry
                ck = candD[pl.ds(t * 16, 16)]
                cv = candI[pl.ds(t * 16, 16)]
                ck, cv = plsc.sort_key_val(ck, cv)
                rck = lax.rev(ck, (0,))
                rcv = lax.rev(cv, (0,))
                m = rck < r1k
                sk = jnp.where(m, rck, r1k)
                sv = jnp.where(m, rcv, r1v)
                sk, sv = plsc.sort_key_val(sk, sv)
                rsk = lax.rev(sk, (0,))
                rsv = lax.rev(sv, (0,))
                m2 = r0k < rsk
                lk = jnp.where(m2, r0k, rsk)
                lv = jnp.where(m2, r0v, rsv)
                hk = jnp.where(m2, rsk, r0k)
                hv = jnp.where(m2, rsv, r0v)
                r0k, r0v = plsc.sort_key_val(lk, lv)
                r1k, r1v = plsc.sort_key_val(hk, hv)
                return r0k, r0v, r1k, r1v

            nv = (off + 15) // 16
            _, r0v, _, r1v = lax.fori_loop(
                0, nv, merge_body, (inf16, zero16i, inf16, zero16i))
            obuf[pl.ds(q * _K, 16)] = r0v + gbase
            obuf[pl.ds(q * _K + 16, 16)] = r1v + gbase

        lax.fori_loop(0, _QPW // 2, per_pair, 0, unroll=False)
        pltpu.sync_copy(obuf, out_hbm.at[pl.ds(wid * (_QPW * _K), _QPW * _K)])

    return knn


def _mlp1_body(x_ref, nc_ref, cf_ref, wc_ref, wa_ref, ws_ref,
               u1_ref, i_ref, s1_ref, si_ref):
    step = pl.program_id(0)

    @pl.when(step == 0)
    def _init():
        s1_ref[...] = jnp.zeros_like(s1_ref)
        si_ref[...] = jnp.zeros_like(si_ref)

    u = jnp.dot(x_ref[...], wc_ref[...], preferred_element_type=jnp.float32)
    c = jnp.dot(nc_ref[...], wa_ref[...], preferred_element_type=jnp.float32)
    u = (u.reshape(_TG, _K, u.shape[-1]) - c[:, None, :]).reshape(u.shape)
    u1_ref[...] = u
    it = jnp.dot(cf_ref[...], ws_ref[...], preferred_element_type=jnp.float32)
    i_ref[...] = it
    s1_ref[0:1, :] += jnp.sum(u, axis=0, keepdims=True)
    s1_ref[1:2, :] += jnp.sum(u * u, axis=0, keepdims=True)
    si_ref[0:1, :] += jnp.sum(it, axis=0, keepdims=True)
    si_ref[1:2, :] += jnp.sum(it * it, axis=0, keepdims=True)


def _mlp2_body(u1_ref, s1_ref, w2_ref, gb1_ref, n_ref, r_ref, s2_ref):
    step = pl.program_id(0)

    @pl.when(step == 0)
    def _init():
        s2_ref[...] = jnp.zeros_like(s2_ref)

    n = n_ref[0]
    mean = s1_ref[0:1, :] / n
    var = s1_ref[1:2, :] / n - mean * mean
    inv = lax.rsqrt(var + _EPS)
    scale = gb1_ref[0:1, :] * inv
    shift = gb1_ref[1:2, :] - mean * scale
    h = jnp.maximum(u1_ref[...] * scale + shift, 0.0)
    u2 = jnp.dot(h, w2_ref[...], preferred_element_type=jnp.float32)
    s2_ref[0:1, :] += jnp.sum(u2, axis=0, keepdims=True)
    s2_ref[1:2, :] += jnp.sum(u2 * u2, axis=0, keepdims=True)
    r_ref[...] = jnp.max(u2.reshape(_TG, _K, u2.shape[-1]), axis=1)


def _final_body(r_ref, i_ref, s2_ref, si_ref, p_ref, n_ref, o_ref):
    n2 = n_ref[0]
    ni = n_ref[1]
    m2 = s2_ref[0:1, :] / n2
    v2 = s2_ref[1:2, :] / n2 - m2 * m2
    inv2 = lax.rsqrt(v2 + _EPS)
    sc2 = p_ref[0:1, :] * inv2
    sh2 = p_ref[1:2, :] - m2 * sc2
    mi = si_ref[0:1, :] / ni
    vi = si_ref[1:2, :] / ni - mi * mi
    invi = lax.rsqrt(vi + _EPS)
    sci = p_ref[2:3, :] * invi
    shi = p_ref[3:4, :] - mi * sci
    red = jnp.maximum(r_ref[...] * sc2 + sh2, 0.0)
    idn = i_ref[...] * sci + shi
    o_ref[...] = jnp.maximum(red + idn, 0.0)


def kernel(coords, feats, W1, g1, b1, W2, g2, b2, Ws, gs, bs):
    B, N, _ = coords.shape
    M = N // 4
    C = feats.shape[1]
    MID = W1.shape[0]
    OUT = W2.shape[0]
    R = B * M * _K
    NG = B * M

    P = jnp.transpose(coords, (2, 0, 1)).reshape(3 * B, N)
    fps_idx = pl.pallas_call(
        functools.partial(_fps_body, n_samples=M),
        grid=(1,),
        in_specs=[pl.BlockSpec((3 * B, N), lambda i: (0, 0))],
        out_specs=pl.BlockSpec((B, M), lambda i: (0, 0)),
        out_shape=jax.ShapeDtypeStruct((B, M), jnp.int32),
        scratch_shapes=[pltpu.VMEM((B, N), jnp.float32)],
    )(P)

    featsT = jnp.transpose(feats, (0, 2, 1))
    pad = 80 - (3 + C)
    table = jnp.concatenate(
        [coords, featsT, jnp.zeros((B, N, pad), jnp.float32)], axis=-1
    ).reshape(B * N, 3 + C + pad)

    base = (jnp.arange(B, dtype=jnp.int32) * N)[:, None]
    cf_idx = (fps_idx + base).reshape(-1)
    new_coords = jnp.take(coords.reshape(B * N, 3), cf_idx, axis=0).reshape(B, M, 3)

    q2f = jnp.sum(new_coords * new_coords, axis=-1).reshape(NG)
    s2 = jnp.sum(coords * coords, axis=-1)
    coordsB = jnp.transpose(coords, (0, 2, 1))
    Qmat = jnp.concatenate([
        new_coords.reshape(NG, 3),
        jnp.zeros((NG, 5), jnp.float32)], axis=1)
    Smat = jnp.concatenate([
        coordsB,
        jnp.zeros((B, 5, N), jnp.float32)], axis=1)
    taus = jnp.array(_TAUS, jnp.float32)
    t12 = taus[None, :] - q2f[:, None]
    t_in = jnp.concatenate([t12, jnp.zeros((NG, 116), jnp.float32)], axis=1)
    d2mat, cnt = pl.pallas_call(
        _score_body,
        grid=(B,),
        in_specs=[
            pl.BlockSpec((M, 8), lambda i: (i, 0)),
            pl.BlockSpec((1, 8, N), lambda i: (i, 0, 0)),
            pl.BlockSpec((1, 1, N), lambda i: (i, 0, 0)),
            pl.BlockSpec((M, 128), lambda i: (i, 0)),
        ],
        out_specs=[
            pl.BlockSpec((M, N), lambda i: (i, 0)),
            pl.BlockSpec((M, 128), lambda i: (i, 0)),
        ],
        out_shape=[
            jax.ShapeDtypeStruct((NG, N), jnp.float32),
            jax.ShapeDtypeStruct((NG, 128), jnp.float32),
        ],
    )(Qmat, Smat, s2[:, None, :], t_in)
    tau_sel = jnp.min(jnp.where(cnt[:, :len(_TAUS)] >= 33.0, t12, jnp.inf),
                      axis=1)
    flat_idx = _make_knn_sc(B, N, M, NG)(
        d2mat, tau_sel.reshape(_NW, _QPW))

    x = jnp.take(table, flat_idx, axis=0)
    cfeats = jnp.take(table, cf_idx, axis=0)[:, 3:3 + C]
    nc_pad = jnp.zeros((NG, 8), jnp.float32).at[:, :3].set(
        new_coords.reshape(NG, 3))

    w1a = jnp.transpose(W1[:, :3]) / _RADIUS
    Wc = jnp.zeros((3 + C + pad, MID), jnp.float32)
    Wc = Wc.at[:3].set(w1a).at[3:3 + C].set(jnp.transpose(W1[:, 3:]))
    Wa = jnp.zeros((8, MID), jnp.float32).at[:3].set(w1a)
    W2T = jnp.transpose(W2)
    WsT = jnp.transpose(Ws)

    gb1 = jnp.zeros((8, MID), jnp.float32).at[0].set(g1).at[1].set(b1)
    P = jnp.zeros((8, OUT), jnp.float32).at[0].set(g2).at[1].set(b2)\
        .at[2].set(gs).at[3].set(bs)

    grid_a = R // _TR
    u1, ident, s1, si = pl.pallas_call(
        _mlp1_body,
        grid=(grid_a,),
        in_specs=[
            pl.BlockSpec((_TR, 3 + C + pad), lambda i: (i, 0)),
            pl.BlockSpec((_TG, 8), lambda i: (i, 0)),
            pl.BlockSpec((_TG, C), lambda i: (i, 0)),
            pl.BlockSpec((3 + C + pad, MID), lambda i: (0, 0)),
            pl.BlockSpec((8, MID), lambda i: (0, 0)),
            pl.BlockSpec((C, OUT), lambda i: (0, 0)),
        ],
        out_specs=[
            pl.BlockSpec((_TR, MID), lambda i: (i, 0)),
            pl.BlockSpec((_TG, OUT), lambda i: (i, 0)),
            pl.BlockSpec((8, MID), lambda i: (0, 0)),
            pl.BlockSpec((8, OUT), lambda i: (0, 0)),
        ],
        out_shape=[
            jax.ShapeDtypeStruct((R, MID), jnp.float32),
            jax.ShapeDtypeStruct((NG, OUT), jnp.float32),
            jax.ShapeDtypeStruct((8, MID), jnp.float32),
            jax.ShapeDtypeStruct((8, OUT), jnp.float32),
        ],
    )(x, nc_pad, cfeats, Wc, Wa, WsT)

    counts = jnp.array([float(R), float(NG)], jnp.float32)

    rmax, s2s = pl.pallas_call(
        _mlp2_body,
        grid=(grid_a,),
        in_specs=[
            pl.BlockSpec((_TR, MID), lambda i: (i, 0)),
            pl.BlockSpec((8, MID), lambda i: (0, 0)),
            pl.BlockSpec((MID, OUT), lambda i: (0, 0)),
            pl.BlockSpec((8, MID), lambda i: (0, 0)),
            pl.BlockSpec(memory_space=pltpu.SMEM),
        ],
        out_specs=[
            pl.BlockSpec((_TG, OUT), lambda i: (i, 0)),
            pl.BlockSpec((8, OUT), lambda i: (0, 0)),
        ],
        out_shape=[
            jax.ShapeDtypeStruct((NG, OUT), jnp.float32),
            jax.ShapeDtypeStruct((8, OUT), jnp.float32),
        ],
    )(u1, s1, W2T, gb1, counts)

    out2d = pl.pallas_call(
        _final_body,
        grid=(B,),
        in_specs=[
            pl.BlockSpec((M, OUT), lambda i: (i, 0)),
            pl.BlockSpec((M, OUT), lambda i: (i, 0)),
            pl.BlockSpec((8, OUT), lambda i: (0, 0)),
            pl.BlockSpec((8, OUT), lambda i: (0, 0)),
            pl.BlockSpec((8, OUT), lambda i: (0, 0)),
            pl.BlockSpec(memory_space=pltpu.SMEM),
        ],
        out_specs=pl.BlockSpec((M, OUT), lambda i: (i, 0)),
        out_shape=jax.ShapeDtypeStruct((NG, OUT), jnp.float32),
    )(rmax, ident, s2s, si, P, counts)

    out = jnp.transpose(out2d.reshape(B, M, OUT), (0, 2, 1))
    return new_coords, out
